# Initial kernel scaffold; baseline (speedup 1.0000x reference)
import jax
import jax.numpy as jnp
from jax.experimental import pallas as pl


def kernel(H_v, sizes):
    raise NotImplementedError("write your pallas kernel here")


# speedup vs baseline: 5.9028x; 5.9028x over previous
import jax
import jax.numpy as jnp
from jax.experimental import pallas as pl

_N = 32640
_D = 512
_B = 256
_R = 320


def _body(h_ref, st_ref, en_ref, out_ref):
    i = pl.program_id(0)

    @pl.when(i == 0)
    def _():
        out_ref[...] = jnp.zeros_like(out_ref)

    r = jax.lax.broadcasted_iota(jnp.int32, (_R, _B), 0) + i * _R
    s = ((r >= st_ref[...]) & (r < en_ref[...])).astype(jnp.float32)
    out_ref[...] += jax.lax.dot_general(
        s, h_ref[...], (((0,), (0,)), ((), ())),
        preferred_element_type=jnp.float32)


def kernel(H_v, sizes):
    offsets = jnp.concatenate(
        [jnp.zeros((1,), jnp.int32), jnp.cumsum(sizes, dtype=jnp.int32)])
    starts = offsets[:-1].reshape(1, _B)
    ends = offsets[1:].reshape(1, _B)
    grid = _N // _R
    return pl.pallas_call(
        _body,
        grid=(grid,),
        in_specs=[
            pl.BlockSpec((_R, _D), lambda i: (i, 0)),
            pl.BlockSpec((1, _B), lambda i: (0, 0)),
            pl.BlockSpec((1, _B), lambda i: (0, 0)),
        ],
        out_specs=pl.BlockSpec((_B, _D), lambda i: (0, 0)),
        out_shape=jax.ShapeDtypeStruct((_B, _D), jnp.float32),
    )(H_v, starts, ends)


# speedup vs baseline: 5.9577x; 1.0093x over previous
import jax
import jax.numpy as jnp
from jax.experimental import pallas as pl

_N = 32640
_D = 512
_B = 256
_R = 320


def _body(h_ref, st_ref, en_ref, out_ref):
    i = pl.program_id(0)

    @pl.when(i == 0)
    def _():
        out_ref[...] = jnp.zeros_like(out_ref)

    r = jax.lax.broadcasted_iota(jnp.int32, (_R, _B), 0) + i * _R
    s = ((r >= st_ref[...]) & (r < en_ref[...])).astype(jnp.bfloat16)
    out_ref[...] += jax.lax.dot_general(
        s, h_ref[...].astype(jnp.bfloat16), (((0,), (0,)), ((), ())),
        preferred_element_type=jnp.float32)


def kernel(H_v, sizes):
    offsets = jnp.concatenate(
        [jnp.zeros((1,), jnp.int32), jnp.cumsum(sizes, dtype=jnp.int32)])
    starts = offsets[:-1].reshape(1, _B)
    ends = offsets[1:].reshape(1, _B)
    grid = _N // _R
    return pl.pallas_call(
        _body,
        grid=(grid,),
        in_specs=[
            pl.BlockSpec((_R, _D), lambda i: (i, 0)),
            pl.BlockSpec((1, _B), lambda i: (0, 0)),
            pl.BlockSpec((1, _B), lambda i: (0, 0)),
        ],
        out_specs=pl.BlockSpec((_B, _D), lambda i: (0, 0)),
        out_shape=jax.ShapeDtypeStruct((_B, _D), jnp.float32),
    )(H_v, starts, ends)


# speedup vs baseline: 12.4275x; 2.0860x over previous
import jax
import jax.numpy as jnp
from jax.experimental import pallas as pl

_N = 32640
_D = 512
_B = 256
_R = 1088


def _body(h_ref, st_ref, en_ref, out_ref):
    i = pl.program_id(0)

    @pl.when(i == 0)
    def _():
        out_ref[...] = jnp.zeros_like(out_ref)

    r = jax.lax.broadcasted_iota(jnp.int32, (_R, _B), 0) + i * _R
    s = ((r >= st_ref[...]) & (r < en_ref[...])).astype(jnp.bfloat16)
    out_ref[...] += jax.lax.dot_general(
        s, h_ref[...].astype(jnp.bfloat16), (((0,), (0,)), ((), ())),
        preferred_element_type=jnp.float32)


def kernel(H_v, sizes):
    offsets = jnp.concatenate(
        [jnp.zeros((1,), jnp.int32), jnp.cumsum(sizes, dtype=jnp.int32)])
    starts = offsets[:-1].reshape(1, _B)
    ends = offsets[1:].reshape(1, _B)
    grid = _N // _R
    return pl.pallas_call(
        _body,
        grid=(grid,),
        in_specs=[
            pl.BlockSpec((_R, _D), lambda i: (i, 0)),
            pl.BlockSpec((1, _B), lambda i: (0, 0)),
            pl.BlockSpec((1, _B), lambda i: (0, 0)),
        ],
        out_specs=pl.BlockSpec((_B, _D), lambda i: (0, 0)),
        out_shape=jax.ShapeDtypeStruct((_B, _D), jnp.float32),
    )(H_v, starts, ends)


# speedup vs baseline: 18.1518x; 1.4606x over previous
import jax
import jax.numpy as jnp
from jax.experimental import pallas as pl

_N = 32640
_D = 512
_B = 256
_R = 3264


def _body(h_ref, st_ref, en_ref, out_ref):
    i = pl.program_id(0)

    @pl.when(i == 0)
    def _():
        out_ref[...] = jnp.zeros_like(out_ref)

    r = jax.lax.broadcasted_iota(jnp.int32, (_R, _B), 0) + i * _R
    s = ((r >= st_ref[...]) & (r < en_ref[...])).astype(jnp.bfloat16)
    out_ref[...] += jax.lax.dot_general(
        s, h_ref[...].astype(jnp.bfloat16), (((0,), (0,)), ((), ())),
        preferred_element_type=jnp.float32)


def kernel(H_v, sizes):
    offsets = jnp.concatenate(
        [jnp.zeros((1,), jnp.int32), jnp.cumsum(sizes, dtype=jnp.int32)])
    starts = offsets[:-1].reshape(1, _B)
    ends = offsets[1:].reshape(1, _B)
    grid = _N // _R
    return pl.pallas_call(
        _body,
        grid=(grid,),
        in_specs=[
            pl.BlockSpec((_R, _D), lambda i: (i, 0)),
            pl.BlockSpec((1, _B), lambda i: (0, 0)),
            pl.BlockSpec((1, _B), lambda i: (0, 0)),
        ],
        out_specs=pl.BlockSpec((_B, _D), lambda i: (0, 0)),
        out_shape=jax.ShapeDtypeStruct((_B, _D), jnp.float32),
    )(H_v, starts, ends)
